# Initial kernel scaffold; baseline (speedup 1.0000x reference)
import jax
import jax.numpy as jnp
from jax.experimental import pallas as pl


def kernel(x, edge_index, W, b):
    raise NotImplementedError("write your pallas kernel here")


# speedup vs baseline: 9.2309x; 9.2309x over previous
import functools

import jax
import jax.numpy as jnp
from jax import lax
from jax.experimental import pallas as pl
from jax.experimental.pallas import tpu as pltpu
from jax.experimental.pallas import tpu_sc as plsc

N = 10000
NP = 10240
E = 320000
D = 128
C = 40
CP = 64
NW = 32
CPW = CP // NW
EPW = E // NW
NB = 512
CHUNK = 4000
L = 16

_sc_mesh = plsc.VectorSubcoreMesh(core_axis_name="c", subcore_axis_name="s")
_sc_params = pltpu.CompilerParams(needs_layout_passes=False)


def _wid():
    return lax.axis_index("c") * 16 + lax.axis_index("s")


@functools.partial(
    pl.kernel,
    out_type=jax.ShapeDtypeStruct((NW, NP), jnp.float32),
    mesh=_sc_mesh,
    compiler_params=_sc_params,
    scratch_types=[
        pltpu.VMEM((EPW,), jnp.int32),
        pltpu.VMEM((NP,), jnp.float32),
    ],
)
def _deg_parts(dst_hbm, parts_hbm, dbuf, acc):
    wid = _wid()
    zeros = jnp.zeros((L,), jnp.float32)

    def zbody(i, carry):
        acc[pl.ds(i * L, L)] = zeros
        return carry

    lax.fori_loop(0, NP // L, zbody, 0)

    pltpu.sync_copy(dst_hbm.at[pl.ds(wid * EPW, EPW)], dbuf)
    ones = jnp.ones((L,), jnp.float32)

    def body(i, carry):
        d16 = dbuf[pl.ds(i * L, L)]
        plsc.addupdate_scatter(acc, [d16], ones)
        return carry

    lax.fori_loop(0, EPW // L, body, 0)
    pltpu.sync_copy(acc, parts_hbm.at[wid])


def _edge_loop(src_hbm, dst_hbm, sbuf, dbuf, inner):
    def chunk_body(k, carry):
        pltpu.sync_copy(src_hbm.at[pl.ds(k * CHUNK, CHUNK)], sbuf)
        pltpu.sync_copy(dst_hbm.at[pl.ds(k * CHUNK, CHUNK)], dbuf)

        def body(j, c2):
            s16 = sbuf[pl.ds(j * L, L)]
            d16 = dbuf[pl.ds(j * L, L)]
            inner(s16, d16)
            return c2

        lax.fori_loop(0, CHUNK // L, body, 0)
        return carry

    lax.fori_loop(0, E // CHUNK, chunk_body, 0)


@functools.partial(
    pl.kernel,
    out_type=jax.ShapeDtypeStruct((CP, NP), jnp.float32),
    mesh=_sc_mesh,
    compiler_params=_sc_params,
    scratch_types=[
        pltpu.VMEM((CPW, NP), jnp.float32),
        pltpu.VMEM((CPW, NP), jnp.float32),
        pltpu.VMEM((CHUNK,), jnp.int32),
        pltpu.VMEM((CHUNK,), jnp.int32),
    ],
)
def _hop1(y_hbm, src_hbm, dst_hbm, z_hbm, ybuf, acc, sbuf, dbuf):
    c0 = _wid() * CPW
    pltpu.sync_copy(y_hbm.at[pl.ds(c0, CPW)], ybuf)
    pltpu.sync_copy(y_hbm.at[pl.ds(c0, CPW)], acc)
    cols = [jnp.full((L,), ci, jnp.int32) for ci in range(CPW)]

    def inner(s16, d16):
        for ci in range(CPW):
            v = plsc.load_gather(ybuf, [cols[ci], s16])
            plsc.addupdate_scatter(acc, [cols[ci], d16], v)

    _edge_loop(src_hbm, dst_hbm, sbuf, dbuf, inner)
    pltpu.sync_copy(acc, z_hbm.at[pl.ds(c0, CPW)])


@functools.partial(
    pl.kernel,
    out_type=jax.ShapeDtypeStruct((CP, NP), jnp.float32),
    mesh=_sc_mesh,
    compiler_params=_sc_params,
    scratch_types=[
        pltpu.VMEM((CPW, NP), jnp.float32),
        pltpu.VMEM((CPW, NP), jnp.float32),
        pltpu.VMEM((NP,), jnp.float32),
        pltpu.VMEM((CHUNK,), jnp.int32),
        pltpu.VMEM((CHUNK,), jnp.int32),
    ],
)
def _hop2(z1_hbm, src_hbm, dst_hbm, dis2_hbm, z_hbm, zbuf, acc, d2buf, sbuf, dbuf):
    c0 = _wid() * CPW
    pltpu.sync_copy(z1_hbm.at[pl.ds(c0, CPW)], zbuf)
    pltpu.sync_copy(dis2_hbm, d2buf)
    def init_body(i, carry):
        w = d2buf[pl.ds(i * L, L)]
        for ci in range(CPW):
            acc[ci, pl.ds(i * L, L)] = zbuf[ci, pl.ds(i * L, L)] * w
        return carry

    lax.fori_loop(0, NP // L, init_body, 0)
    cols = [jnp.full((L,), ci, jnp.int32) for ci in range(CPW)]

    def inner(s16, d16):
        g = plsc.load_gather(d2buf, [s16])
        for ci in range(CPW):
            v = plsc.load_gather(zbuf, [cols[ci], s16]) * g
            plsc.addupdate_scatter(acc, [cols[ci], d16], v)

    _edge_loop(src_hbm, dst_hbm, sbuf, dbuf, inner)
    pltpu.sync_copy(acc, z_hbm.at[pl.ds(c0, CPW)])


def _tc1_body(x_ref, wt_ref, parts_ref, y_ref, dis_ref, dis2_ref):
    deg = 1.0 + jnp.sum(parts_ref[...], axis=0, keepdims=True)
    dis = lax.rsqrt(deg)
    yt = lax.dot_general(
        wt_ref[...], x_ref[...], (((1,), (1,)), ((), ())),
        preferred_element_type=jnp.float32,
    )
    y_ref[...] = yt * dis
    dis_ref[...] = dis
    dis2_ref[...] = 1.0 / deg


def _tc1(x_pad, wt, parts):
    return pl.pallas_call(
        _tc1_body,
        grid=(NP // NB,),
        in_specs=[
            pl.BlockSpec((NB, D), lambda i: (i, 0)),
            pl.BlockSpec((CP, D), lambda i: (0, 0)),
            pl.BlockSpec((NW, NB), lambda i: (0, i)),
        ],
        out_specs=[
            pl.BlockSpec((CP, NB), lambda i: (0, i)),
            pl.BlockSpec((1, NB), lambda i: (0, i)),
            pl.BlockSpec((1, NB), lambda i: (0, i)),
        ],
        out_shape=[
            jax.ShapeDtypeStruct((CP, NP), jnp.float32),
            jax.ShapeDtypeStruct((1, NP), jnp.float32),
            jax.ShapeDtypeStruct((1, NP), jnp.float32),
        ],
    )(x_pad, wt, parts)


def _tc3_body(z_ref, dis_ref, b_ref, o_ref):
    o = (z_ref[...] * dis_ref[...]).T + b_ref[...]
    m = jnp.max(o, axis=1, keepdims=True)
    lse = m + jnp.log(jnp.sum(jnp.exp(o - m), axis=1, keepdims=True))
    o_ref[...] = o - lse


def _tc3(z2, dis, b_pad):
    return pl.pallas_call(
        _tc3_body,
        grid=(NP // NB,),
        in_specs=[
            pl.BlockSpec((CP, NB), lambda i: (0, i)),
            pl.BlockSpec((1, NB), lambda i: (0, i)),
            pl.BlockSpec((1, CP), lambda i: (0, 0)),
        ],
        out_specs=pl.BlockSpec((NB, CP), lambda i: (i, 0)),
        out_shape=jax.ShapeDtypeStruct((NP, CP), jnp.float32),
    )(z2, dis, b_pad)


@jax.jit
def kernel(x, edge_index, W, b):
    src = edge_index[0].astype(jnp.int32)
    dst = edge_index[1].astype(jnp.int32)
    x_pad = jnp.zeros((NP, D), jnp.float32).at[:N].set(x)
    wt = jnp.zeros((CP, D), jnp.float32).at[:C].set(W.T)
    b_pad = jnp.full((1, CP), -1e30, jnp.float32).at[0, :C].set(b)

    parts = _deg_parts(dst)
    y0, dis, dis2 = _tc1(x_pad, wt, parts)
    z1 = _hop1(y0, src, dst)
    z2 = _hop2(z1, src, dst, dis2.reshape(NP))
    out = _tc3(z2, dis, b_pad)
    return out[:N, :C]


# speedup vs baseline: 10.4144x; 1.1282x over previous
import functools

import jax
import jax.numpy as jnp
from jax import lax
from jax.experimental import pallas as pl
from jax.experimental.pallas import tpu as pltpu
from jax.experimental.pallas import tpu_sc as plsc

N = 10000
NP = 10240
E = 320000
D = 128
C = 40
CP = 64
NW = 32
CPW = CP // NW
EPW = E // NW
NB = 512
CHUNK = 12800
L = 16

_sc_mesh = plsc.VectorSubcoreMesh(core_axis_name="c", subcore_axis_name="s")
_sc_params = pltpu.CompilerParams(needs_layout_passes=False)


def _wid():
    return lax.axis_index("c") * 16 + lax.axis_index("s")


@functools.partial(
    pl.kernel,
    out_type=jax.ShapeDtypeStruct((NW, NP), jnp.float32),
    mesh=_sc_mesh,
    compiler_params=_sc_params,
    scratch_types=[
        pltpu.VMEM((EPW,), jnp.int32),
        pltpu.VMEM((NP,), jnp.float32),
    ],
)
def _deg_parts(dst_hbm, parts_hbm, dbuf, acc):
    wid = _wid()
    zeros = jnp.zeros((L,), jnp.float32)

    def zbody(i, carry):
        acc[pl.ds(i * L, L)] = zeros
        return carry

    lax.fori_loop(0, NP // L, zbody, 0)

    pltpu.sync_copy(dst_hbm.at[pl.ds(wid * EPW, EPW)], dbuf)
    ones = jnp.ones((L,), jnp.float32)

    def body(i, carry):
        d16 = dbuf[pl.ds(i * L, L)]
        plsc.addupdate_scatter(acc, [d16], ones)
        return carry

    lax.fori_loop(0, EPW // L, body, 0)
    pltpu.sync_copy(acc, parts_hbm.at[wid])


UNROLL = 8


def _edge_loop(src_hbm, dst_hbm, sbuf, dbuf, inner):
    def chunk_body(k, carry):
        pltpu.sync_copy(src_hbm.at[pl.ds(k * CHUNK, CHUNK)], sbuf)
        pltpu.sync_copy(dst_hbm.at[pl.ds(k * CHUNK, CHUNK)], dbuf)

        def body(j, c2):
            for u in range(UNROLL):
                s16 = sbuf[pl.ds(j * (L * UNROLL) + u * L, L)]
                d16 = dbuf[pl.ds(j * (L * UNROLL) + u * L, L)]
                inner(s16, d16)
            return c2

        lax.fori_loop(0, CHUNK // (L * UNROLL), body, 0)
        return carry

    lax.fori_loop(0, E // CHUNK, chunk_body, 0)


@functools.partial(
    pl.kernel,
    out_type=jax.ShapeDtypeStruct((CP, NP), jnp.float32),
    mesh=_sc_mesh,
    compiler_params=_sc_params,
    scratch_types=[
        pltpu.VMEM((CPW, NP), jnp.float32),
        pltpu.VMEM((CPW, NP), jnp.float32),
        pltpu.VMEM((CHUNK,), jnp.int32),
        pltpu.VMEM((CHUNK,), jnp.int32),
    ],
)
def _hop1(y_hbm, src_hbm, dst_hbm, z_hbm, ybuf, acc, sbuf, dbuf):
    c0 = _wid() * CPW
    pltpu.sync_copy(y_hbm.at[pl.ds(c0, CPW)], ybuf)
    pltpu.sync_copy(y_hbm.at[pl.ds(c0, CPW)], acc)
    cols = [jnp.full((L,), ci, jnp.int32) for ci in range(CPW)]

    def inner(s16, d16):
        for ci in range(CPW):
            v = plsc.load_gather(ybuf, [cols[ci], s16])
            plsc.addupdate_scatter(acc, [cols[ci], d16], v)

    _edge_loop(src_hbm, dst_hbm, sbuf, dbuf, inner)
    pltpu.sync_copy(acc, z_hbm.at[pl.ds(c0, CPW)])


@functools.partial(
    pl.kernel,
    out_type=jax.ShapeDtypeStruct((CP, NP), jnp.float32),
    mesh=_sc_mesh,
    compiler_params=_sc_params,
    scratch_types=[
        pltpu.VMEM((CPW, NP), jnp.float32),
        pltpu.VMEM((CPW, NP), jnp.float32),
        pltpu.VMEM((NP,), jnp.float32),
        pltpu.VMEM((CHUNK,), jnp.int32),
        pltpu.VMEM((CHUNK,), jnp.int32),
    ],
)
def _hop2(z1_hbm, src_hbm, dst_hbm, dis2_hbm, z_hbm, zbuf, acc, d2buf, sbuf, dbuf):
    c0 = _wid() * CPW
    pltpu.sync_copy(z1_hbm.at[pl.ds(c0, CPW)], zbuf)
    pltpu.sync_copy(dis2_hbm, d2buf)
    def init_body(i, carry):
        w = d2buf[pl.ds(i * L, L)]
        for ci in range(CPW):
            acc[ci, pl.ds(i * L, L)] = zbuf[ci, pl.ds(i * L, L)] * w
        return carry

    lax.fori_loop(0, NP // L, init_body, 0)
    cols = [jnp.full((L,), ci, jnp.int32) for ci in range(CPW)]

    def inner(s16, d16):
        g = plsc.load_gather(d2buf, [s16])
        for ci in range(CPW):
            v = plsc.load_gather(zbuf, [cols[ci], s16]) * g
            plsc.addupdate_scatter(acc, [cols[ci], d16], v)

    _edge_loop(src_hbm, dst_hbm, sbuf, dbuf, inner)
    pltpu.sync_copy(acc, z_hbm.at[pl.ds(c0, CPW)])


def _tc1_body(x_ref, wt_ref, parts_ref, y_ref, dis_ref, dis2_ref):
    deg = 1.0 + jnp.sum(parts_ref[...], axis=0, keepdims=True)
    dis = lax.rsqrt(deg)
    yt = lax.dot_general(
        wt_ref[...], x_ref[...], (((1,), (1,)), ((), ())),
        preferred_element_type=jnp.float32,
    )
    y_ref[...] = yt * dis
    dis_ref[...] = dis
    dis2_ref[...] = 1.0 / deg


def _tc1(x_pad, wt, parts):
    return pl.pallas_call(
        _tc1_body,
        grid=(NP // NB,),
        in_specs=[
            pl.BlockSpec((NB, D), lambda i: (i, 0)),
            pl.BlockSpec((CP, D), lambda i: (0, 0)),
            pl.BlockSpec((NW, NB), lambda i: (0, i)),
        ],
        out_specs=[
            pl.BlockSpec((CP, NB), lambda i: (0, i)),
            pl.BlockSpec((1, NB), lambda i: (0, i)),
            pl.BlockSpec((1, NB), lambda i: (0, i)),
        ],
        out_shape=[
            jax.ShapeDtypeStruct((CP, NP), jnp.float32),
            jax.ShapeDtypeStruct((1, NP), jnp.float32),
            jax.ShapeDtypeStruct((1, NP), jnp.float32),
        ],
    )(x_pad, wt, parts)


def _tc3_body(z_ref, dis_ref, b_ref, o_ref):
    o = (z_ref[...] * dis_ref[...]).T + b_ref[...]
    m = jnp.max(o, axis=1, keepdims=True)
    lse = m + jnp.log(jnp.sum(jnp.exp(o - m), axis=1, keepdims=True))
    o_ref[...] = o - lse


def _tc3(z2, dis, b_pad):
    return pl.pallas_call(
        _tc3_body,
        grid=(NP // NB,),
        in_specs=[
            pl.BlockSpec((CP, NB), lambda i: (0, i)),
            pl.BlockSpec((1, NB), lambda i: (0, i)),
            pl.BlockSpec((1, CP), lambda i: (0, 0)),
        ],
        out_specs=pl.BlockSpec((NB, CP), lambda i: (i, 0)),
        out_shape=jax.ShapeDtypeStruct((NP, CP), jnp.float32),
    )(z2, dis, b_pad)


@jax.jit
def kernel(x, edge_index, W, b):
    src = edge_index[0].astype(jnp.int32)
    dst = edge_index[1].astype(jnp.int32)
    x_pad = jnp.zeros((NP, D), jnp.float32).at[:N].set(x)
    wt = jnp.zeros((CP, D), jnp.float32).at[:C].set(W.T)
    b_pad = jnp.full((1, CP), -1e30, jnp.float32).at[0, :C].set(b)

    parts = _deg_parts(dst)
    y0, dis, dis2 = _tc1(x_pad, wt, parts)
    z1 = _hop1(y0, src, dst)
    z2 = _hop2(z1, src, dst, dis2.reshape(NP))
    out = _tc3(z2, dis, b_pad)
    return out[:N, :C]
